# baseline (device time: 76724 ns/iter reference)
import jax
import jax.numpy as jnp
from jax import lax
from jax.experimental import pallas as pl
from jax.experimental.pallas import tpu as pltpu

N_DEV = 4


def kernel(x, w_mat, scale_x, scale_w):
    m_per, k = x.shape
    _, n_per = w_mat.shape
    k_half = k // 2
    m_half = m_per // 2
    KTOP = pl.ds(0, k_half)
    KBOT = pl.ds(k_half, k_half)
    MTOP = pl.ds(0, m_half)
    MBOT = pl.ds(m_half, m_half)

    def body(x_hbm, w_hbm, sx_ref, sw_ref, out_hbm,
             xv, wv, x8, comm_w, blk_send, fr, fl, blk_recv, acc,
             snd, rcv, lsem, osem):
        my = lax.axis_index("i")
        left = lax.rem(my + (N_DEV - 1), N_DEV)
        right = lax.rem(my + 1, N_DEV)
        opp = lax.rem(my + 2, N_DEV)

        k_q = k // 4
        KQ = [pl.ds(i * k_q, k_q) for i in range(4)]

        def loadw(sl, i):
            cp = pltpu.make_async_copy(
                w_hbm.at[sl, :], wv.at[sl, :], lsem.at[i])
            cp.start()
            return cp

        cp_w0 = loadw(KQ[0], 0)
        cp_w3 = loadw(KQ[3], 3)
        cp_w1 = loadw(KQ[1], 1)
        cp_w2 = loadw(KQ[2], 2)

        barrier_sem = pltpu.get_barrier_semaphore()
        for nbr in (left, right):
            pl.semaphore_signal(
                barrier_sem, inc=1,
                device_id=(nbr,), device_id_type=pl.DeviceIdType.MESH,
            )
        pl.semaphore_wait(barrier_sem, 2)

        def rc(src, dst, i, tgt):
            r_ = pltpu.make_async_remote_copy(
                src_ref=src, dst_ref=dst,
                send_sem=snd.at[i], recv_sem=rcv.at[i],
                device_id=(tgt,), device_id_type=pl.DeviceIdType.MESH,
            )
            r_.start()
            return r_

        def castw(sl):
            comm_w[0, sl, :] = wv[sl, :].astype(jnp.float8_e5m2)

        cp_w0.wait()
        castw(KQ[0])
        wr_q0 = rc(comm_w.at[0, KQ[0]], comm_w.at[1, KQ[0]], 0, right)
        cp_w3.wait()
        castw(KQ[3])
        wl_q3 = rc(comm_w.at[0, KQ[3]], comm_w.at[2, KQ[3]], 3, left)
        cp_w1.wait()
        castw(KQ[1])
        wr_q1 = rc(comm_w.at[0, KQ[1]], comm_w.at[1, KQ[1]], 1, right)
        cp_w2.wait()
        castw(KQ[2])
        wl_q2 = rc(comm_w.at[0, KQ[2]], comm_w.at[2, KQ[2]], 4, left)
        wr_tl = rc(comm_w.at[0, KBOT], comm_w.at[1, KBOT], 2, right)
        wl_tl = rc(comm_w.at[0, KTOP], comm_w.at[2, KTOP], 5, left)

        cp_x = pltpu.make_async_copy(x_hbm, xv, lsem.at[4])
        cp_x.start()
        cp_x.wait()
        x8[...] = xv[...].astype(jnp.float8_e5m2)
        scale = sx_ref[0] * sw_ref[0]

        def gemm(w_chunk):
            y = lax.dot_general(
                x8[...], w_chunk,
                (((1,), (0,)), ((), ())),
                preferred_element_type=jnp.float32,
            )
            return jnp.maximum(y * scale, 0.0)

        out_cps = []

        def store_out(rows_val, origin, row_off, rows, osem_i):
            sl = pl.ds(origin * m_per + row_off, rows)
            acc[sl, :] = rows_val
            cp = pltpu.make_async_copy(acc.at[sl, :], out_hbm.at[sl, :],
                                       osem.at[osem_i])
            cp.start()
            out_cps.append(cp)

        store_out(gemm(comm_w[0]), my, 0, m_per, 0)

        wr_q0.wait_recv()
        wr_q1.wait_recv()
        fw_r = rc(comm_w.at[1, KTOP], comm_w.at[3, KTOP], 6, right)
        wl_q3.wait_recv()
        wl_q2.wait_recv()
        fw_l = rc(comm_w.at[2, KBOT], comm_w.at[3, KBOT], 7, left)

        wr_tl.wait_recv()
        blk_send[1] = gemm(comm_w[1]).astype(jnp.bfloat16)
        b_l = rc(blk_send.at[1], blk_recv.at[1], 9, left)
        wl_tl.wait_recv()
        blk_send[0] = gemm(comm_w[2]).astype(jnp.bfloat16)
        b_r = rc(blk_send.at[0], blk_recv.at[0], 8, right)

        m_q = m_per // 4
        MQ = [pl.ds(i * m_q, m_q) for i in range(4)]
        FQ = [pl.ds(0, m_q), pl.ds(m_q, m_q)]
        fw_r.wait_recv()
        fw_l.wait_recv()
        blk_send[2] = gemm(comm_w[3]).astype(jnp.bfloat16)
        d_r0 = rc(blk_send.at[2, MQ[0]], fr.at[FQ[0]], 10, right)
        d_r1 = rc(blk_send.at[2, MQ[1]], fr.at[FQ[1]], 11, right)
        d_l0 = rc(blk_send.at[2, MQ[3]], fl.at[FQ[1]], 12, left)
        d_l1 = rc(blk_send.at[2, MQ[2]], fl.at[FQ[0]], 13, left)

        b_r.wait_recv()
        store_out(blk_recv[0].astype(jnp.float32), left, 0, m_per, 1)
        b_l.wait_recv()
        store_out(blk_recv[1].astype(jnp.float32), right, 0, m_per, 2)

        d_r0.wait_recv()
        f_r0 = rc(fr.at[FQ[0]], blk_recv.at[2, MQ[0]], 14, right)
        d_l0.wait_recv()
        f_l0 = rc(fl.at[FQ[1]], blk_recv.at[2, MQ[3]], 16, left)
        d_r1.wait_recv()
        f_r1 = rc(fr.at[FQ[1]], blk_recv.at[2, MQ[1]], 15, right)
        d_l1.wait_recv()
        f_l1 = rc(fl.at[FQ[0]], blk_recv.at[2, MQ[2]], 17, left)

        f_r0.wait_recv()
        store_out(blk_recv[2, MQ[0]].astype(jnp.float32),
                  opp, 0, m_q, 3)
        f_l0.wait_recv()
        store_out(blk_recv[2, MQ[3]].astype(jnp.float32),
                  opp, 3 * m_q, m_q, 4)
        f_r1.wait_recv()
        store_out(blk_recv[2, MQ[1]].astype(jnp.float32),
                  opp, m_q, m_q, 5)
        f_l1.wait_recv()
        store_out(blk_recv[2, MQ[2]].astype(jnp.float32),
                  opp, 2 * m_q, m_q, 6)

        for cp in out_cps:
            cp.wait()
        for r_ in (wr_q0, wr_q1, wr_tl, wl_q3, wl_q2, wl_tl, fw_r, fw_l,
                   b_l, b_r, d_r0, d_r1, d_l0, d_l1,
                   f_r0, f_r1, f_l0, f_l1):
            r_.wait_send()

    return pl.pallas_call(
        body,
        out_shape=jax.ShapeDtypeStruct((N_DEV * m_per, n_per), jnp.float32),
        in_specs=[
            pl.BlockSpec(memory_space=pl.ANY),
            pl.BlockSpec(memory_space=pl.ANY),
            pl.BlockSpec(memory_space=pltpu.SMEM),
            pl.BlockSpec(memory_space=pltpu.SMEM),
        ],
        out_specs=pl.BlockSpec(memory_space=pl.ANY),
        scratch_shapes=[
            pltpu.VMEM((m_per, k), jnp.float32),
            pltpu.VMEM((k, n_per), jnp.float32),
            pltpu.VMEM((m_per, k), jnp.float8_e5m2),
            pltpu.VMEM((4, k, n_per), jnp.float8_e5m2),
            pltpu.VMEM((3, m_per, n_per), jnp.bfloat16),
            pltpu.VMEM((m_half, n_per), jnp.bfloat16),
            pltpu.VMEM((m_half, n_per), jnp.bfloat16),
            pltpu.VMEM((3, m_per, n_per), jnp.bfloat16),
            pltpu.VMEM((N_DEV * m_per, n_per), jnp.float32),
            pltpu.SemaphoreType.DMA((18,)),
            pltpu.SemaphoreType.DMA((18,)),
            pltpu.SemaphoreType.DMA((5,)),
            pltpu.SemaphoreType.DMA((7,)),
        ],
        input_output_aliases={1: 0},
        compiler_params=pltpu.CompilerParams(
            collective_id=0,
            vmem_limit_bytes=100 * 1024 * 1024,
        ),
    )(x, w_mat, scale_x, scale_w)


# device time: 70087 ns/iter; 1.0947x vs baseline; 1.0947x over previous
import jax
import jax.numpy as jnp
from jax import lax
from jax.experimental import pallas as pl
from jax.experimental.pallas import tpu as pltpu

N_DEV = 4


def kernel(x, w_mat, scale_x, scale_w):
    m_per, k = x.shape
    _, n_per = w_mat.shape
    k_half = k // 2
    m_half = m_per // 2
    KTOP = pl.ds(0, k_half)
    KBOT = pl.ds(k_half, k_half)
    MTOP = pl.ds(0, m_half)
    MBOT = pl.ds(m_half, m_half)

    def body(x_hbm, w_hbm, sx_ref, sw_ref, out_hbm,
             xv, wv, x8, comm_w, blk_send, fr, fl, blk_recv, acc,
             snd, rcv, lsem, osem):
        my = lax.axis_index("i")
        left = lax.rem(my + (N_DEV - 1), N_DEV)
        right = lax.rem(my + 1, N_DEV)
        opp = lax.rem(my + 2, N_DEV)

        k_q = k // 4
        KQ = [pl.ds(i * k_q, k_q) for i in range(4)]

        def loadw(sl, i):
            cp = pltpu.make_async_copy(
                w_hbm.at[sl, :], wv.at[sl, :], lsem.at[i])
            cp.start()
            return cp

        cp_w0 = loadw(KQ[0], 0)
        cp_w3 = loadw(KQ[3], 3)
        cp_w1 = loadw(KQ[1], 1)
        cp_w2 = loadw(KQ[2], 2)

        barrier_sem = pltpu.get_barrier_semaphore()
        for nbr in (left, right):
            pl.semaphore_signal(
                barrier_sem, inc=1,
                device_id=(nbr,), device_id_type=pl.DeviceIdType.MESH,
            )
        pl.semaphore_wait(barrier_sem, 2)

        def rc(src, dst, i, tgt):
            r_ = pltpu.make_async_remote_copy(
                src_ref=src, dst_ref=dst,
                send_sem=snd.at[i], recv_sem=rcv.at[i],
                device_id=(tgt,), device_id_type=pl.DeviceIdType.MESH,
            )
            r_.start()
            return r_

        def castw(sl):
            comm_w[0, sl, :] = wv[sl, :].astype(jnp.float8_e5m2)

        cp_w0.wait()
        castw(KQ[0])
        wr_q0 = rc(comm_w.at[0, KQ[0]], comm_w.at[1, KQ[0]], 0, right)
        cp_w3.wait()
        castw(KQ[3])
        wl_q3 = rc(comm_w.at[0, KQ[3]], comm_w.at[2, KQ[3]], 3, left)
        cp_w1.wait()
        castw(KQ[1])
        wr_q1 = rc(comm_w.at[0, KQ[1]], comm_w.at[1, KQ[1]], 1, right)
        cp_w2.wait()
        castw(KQ[2])
        wl_q2 = rc(comm_w.at[0, KQ[2]], comm_w.at[2, KQ[2]], 4, left)
        wr_tl = rc(comm_w.at[0, KBOT], comm_w.at[1, KBOT], 2, right)
        wl_tl = rc(comm_w.at[0, KTOP], comm_w.at[2, KTOP], 5, left)

        cp_x = pltpu.make_async_copy(x_hbm, xv, lsem.at[4])
        cp_x.start()
        cp_x.wait()
        x8[...] = xv[...].astype(jnp.float8_e5m2)
        scale = sx_ref[0] * sw_ref[0]

        def gemm(w_chunk):
            y = lax.dot_general(
                x8[...], w_chunk,
                (((1,), (0,)), ((), ())),
                preferred_element_type=jnp.float32,
            )
            return jnp.maximum(y * scale, 0.0)

        out_cps = []

        def store_out(rows_val, origin, row_off, rows, osem_i):
            sl = pl.ds(origin * m_per + row_off, rows)
            acc[sl, :] = rows_val
            cp = pltpu.make_async_copy(acc.at[sl, :], out_hbm.at[sl, :],
                                       osem.at[osem_i])
            cp.start()
            out_cps.append(cp)

        store_out(gemm(comm_w[0]), my, 0, m_per, 0)

        wr_q0.wait_recv()
        wr_q1.wait_recv()
        fw_r = rc(comm_w.at[1, KTOP], comm_w.at[3, KTOP], 6, right)
        wl_q3.wait_recv()
        wl_q2.wait_recv()
        fw_l = rc(comm_w.at[2, KBOT], comm_w.at[3, KBOT], 7, left)

        wr_tl.wait_recv()
        blk_send[1] = gemm(comm_w[1]).astype(jnp.bfloat16)
        b_l = rc(blk_send.at[1], blk_recv.at[1], 9, left)
        wl_tl.wait_recv()
        blk_send[0] = gemm(comm_w[2]).astype(jnp.bfloat16)
        b_r = rc(blk_send.at[0], blk_recv.at[0], 8, right)

        m_q = m_per // 4
        MQ = [pl.ds(i * m_q, m_q) for i in range(4)]
        FQ = [pl.ds(0, m_q), pl.ds(m_q, m_q)]
        fw_r.wait_recv()
        fw_l.wait_recv()
        blk_send[2] = gemm(comm_w[3]).astype(jnp.bfloat16)
        d_r0 = rc(blk_send.at[2, MQ[0]], fr.at[FQ[0]], 10, right)
        d_r1 = rc(blk_send.at[2, MQ[1]], fr.at[FQ[1]], 11, right)
        d_l0 = rc(blk_send.at[2, MQ[3]], fl.at[FQ[1]], 12, left)
        d_l1 = rc(blk_send.at[2, MQ[2]], fl.at[FQ[0]], 13, left)

        b_r.wait_recv()
        store_out(blk_recv[0].astype(jnp.float32), left, 0, m_per, 1)
        b_l.wait_recv()
        store_out(blk_recv[1].astype(jnp.float32), right, 0, m_per, 2)

        d_r0.wait_recv()
        f_r0 = rc(fr.at[FQ[0]], blk_recv.at[2, MQ[0]], 14, right)
        d_l0.wait_recv()
        f_l0 = rc(fl.at[FQ[1]], blk_recv.at[2, MQ[3]], 16, left)
        d_r1.wait_recv()
        f_r1 = rc(fr.at[FQ[1]], blk_recv.at[2, MQ[1]], 15, right)
        d_l1.wait_recv()
        f_l1 = rc(fl.at[FQ[0]], blk_recv.at[2, MQ[2]], 17, left)

        f_r0.wait_recv()
        store_out(blk_recv[2, MQ[0]].astype(jnp.float32),
                  opp, 0, m_q, 3)
        f_l0.wait_recv()
        store_out(blk_recv[2, MQ[3]].astype(jnp.float32),
                  opp, 3 * m_q, m_q, 4)
        f_r1.wait_recv()
        store_out(blk_recv[2, MQ[1]].astype(jnp.float32),
                  opp, m_q, m_q, 5)
        f_l1.wait_recv()
        store_out(blk_recv[2, MQ[2]].astype(jnp.float32),
                  opp, 2 * m_q, m_q, 6)

        for cp in out_cps:
            cp.wait()
        for r_ in (wr_q0, wr_q1, wr_tl, wl_q3, wl_q2, wl_tl, fw_r, fw_l,
                   b_l, b_r, d_r0, d_r1, d_l0, d_l1,
                   f_r0, f_r1, f_l0, f_l1):
            r_.wait_send()

    return pl.pallas_call(
        body,
        out_shape=jax.ShapeDtypeStruct((N_DEV * m_per, n_per), jnp.float32),
        in_specs=[
            pl.BlockSpec(memory_space=pl.ANY),
            pl.BlockSpec(memory_space=pl.ANY),
            pl.BlockSpec(memory_space=pltpu.SMEM),
            pl.BlockSpec(memory_space=pltpu.SMEM),
        ],
        out_specs=pl.BlockSpec(memory_space=pl.ANY),
        scratch_shapes=[
            pltpu.VMEM((m_per, k), jnp.float32),
            pltpu.VMEM((k, n_per), jnp.float32),
            pltpu.VMEM((m_per, k), jnp.float8_e5m2),
            pltpu.VMEM((4, k, n_per), jnp.float8_e5m2),
            pltpu.VMEM((3, m_per, n_per), jnp.bfloat16),
            pltpu.VMEM((m_half, n_per), jnp.bfloat16),
            pltpu.VMEM((m_half, n_per), jnp.bfloat16),
            pltpu.VMEM((3, m_per, n_per), jnp.bfloat16),
            pltpu.VMEM((N_DEV * m_per, n_per), jnp.float32),
            pltpu.SemaphoreType.DMA((18,)),
            pltpu.SemaphoreType.DMA((18,)),
            pltpu.SemaphoreType.DMA((5,)),
            pltpu.SemaphoreType.DMA((7,)),
        ],
        compiler_params=pltpu.CompilerParams(
            collective_id=0,
            vmem_limit_bytes=100 * 1024 * 1024,
        ),
    )(x, w_mat, scale_x, scale_w)
